# baseline (speedup 1.0000x reference)
import functools

import jax
import jax.numpy as jnp
from jax import lax
from jax.experimental import pallas as pl
from jax.experimental.pallas import tpu as pltpu
from jax.experimental.pallas import tpu_sc as plsc

IN_F = 512
ASSOC = 50000
OUT_F = 512
TOTAL = IN_F + ASSOC + OUT_F
NC = 2
NS = 16
NW = NC * NS
L = 16
SL = 3200
PT = NS * SL
C = 2048


def _tanh16(v):
  e = jnp.exp(v + v)
  return 1.0 - 2.0 / (e + 1.0)


def _edge_phase(wid, src_h, dst_h, w_h, acc_sh, a_sh, a_loc,
                srcb, wb, dstb, valb, sem_in, sem_sc, n_edges, n_chunks):
  e8 = n_edges // 8
  start = 8 * ((e8 * wid) // NW)
  end = 8 * ((e8 * (wid + 1)) // NW)
  iota = lax.iota(jnp.int32, L)

  def chunk_base(k):
    b0 = start + k * C
    b = jnp.minimum(b0, end - C)
    return b, b0 - b

  def in_descs(k, h):
    b, _ = chunk_base(k)
    return [pltpu.make_async_copy(src_h.at[pl.ds(b, C)], srcb[h],
                                  sem_in.at[h]),
            pltpu.make_async_copy(w_h.at[pl.ds(b, C)], wb[h], sem_in.at[h]),
            pltpu.make_async_copy(dst_h.at[pl.ds(b, C)], dstb[h],
                                  sem_in.at[h])]

  def sc_descs(h):
    return [pltpu.make_async_copy(valb[h], acc_sh.at[dstb[h]], sem_sc.at[h])]

  def compute(k, h):
    _, pinv = chunk_base(k)

    @plsc.parallel_loop(0, C, step=L, unroll=8)
    def g_body(o):
      q = pl.ds(o, L)
      sv = srcb[h][q]
      av = plsc.load_gather(a_loc, [sv])
      wv = jnp.where((o + iota) >= pinv, wb[h][q], 0.0)
      valb[h][q] = av * wv

  def run_chunk(k, h, drain_h):
    if drain_h is not None:
      for d in sc_descs(drain_h):
        d.wait()
    for d in in_descs(k + 1, (h + 1) % 3):
      d.start()
    for d in in_descs(k, h):
      d.wait()
    compute(k, h)
    for d in sc_descs(h):
      d.start(add=True)

  a_copy = pltpu.make_async_copy(a_sh, a_loc, sem_in.at[2])
  a_copy.start()
  for d in in_descs(0, 0):
    d.start()
  a_copy.wait()
  run_chunk(0, 0, None)
  run_chunk(1, 1, None)

  def triple_body(p, carry):
    k = 2 + 3 * p
    for h_off in range(3):
      kk = k + h_off
      h = (2 + h_off) % 3
      run_chunk(kk, h, (h + 1) % 3)
    return carry

  lax.fori_loop(0, (n_chunks - 2) // 3, triple_body, 0)

  for d in sc_descs((n_chunks - 2) % 3):
    d.wait()
  for d in sc_descs((n_chunks - 1) % 3):
    d.wait()
  for d in in_descs(n_chunks, n_chunks % 3):
    d.wait()


def _publish_a(off, acc_sh, a_sh, msto, abuf):
  pltpu.sync_copy(msto, acc_sh.at[pl.ds(off, SL)])
  pltpu.sync_copy(abuf, a_sh.at[pl.ds(off, SL)])
  plsc.subcore_barrier()


def _fused_body(x_h, si_h, di_h, wi_h, src_h, dst_h, w_h, y_h, p1_h,
                acc_sh, a_sh, a_loc, m0, m1, msto, abuf, x_loc,
                sib, dib, wib, vib, srcb, wb, dstb, valb,
                sem_in, sem_sc, sem_x,
                n_in, n_edges, n_chunks):
  c = lax.axis_index("c")
  s = lax.axis_index("s")
  wid = s * NC + c
  off = s * SL

  zeros = jnp.zeros((L,), jnp.float32)

  @plsc.parallel_loop(0, SL, step=L, unroll=4)
  def z_body(o):
    msto[pl.ds(o, L)] = zeros
  pltpu.sync_copy(msto, acc_sh.at[pl.ds(off, SL)])
  pltpu.sync_copy(x_h, x_loc)
  plsc.subcore_barrier()

  per = n_in // NS
  base = s * per
  pltpu.sync_copy(si_h.at[pl.ds(base, per)], sib)
  pltpu.sync_copy(wi_h.at[pl.ds(base, per)], wib)
  pltpu.sync_copy(di_h.at[pl.ds(base, per)], dib)

  @plsc.parallel_loop(0, per, step=L, unroll=4)
  def g_body(o):
    q = pl.ds(o, L)
    sv = sib[q]
    av = plsc.load_gather(x_loc, [sv])
    vib[q] = av * wib[q]
  pltpu.sync_copy(vib, acc_sh.at[dib], add=True)
  plsc.subcore_barrier()

  pltpu.sync_copy(acc_sh.at[pl.ds(off, SL)], m0)
  scale = jnp.where(c == 0, 1.0, 0.0).astype(jnp.float32)

  @plsc.parallel_loop(0, SL, step=L, unroll=4)
  def t_body(o):
    q = pl.ds(o, L)
    m = m0[q]
    abuf[q] = _tanh16(m)
    msto[q] = m * scale
  _publish_a(off, acc_sh, a_sh, msto, abuf)
  _edge_phase(wid, src_h, dst_h, w_h, acc_sh, a_sh, a_loc,
              srcb, wb, dstb, valb, sem_in, sem_sc, n_edges, n_chunks)
  plsc.subcore_barrier()

  pltpu.sync_copy(acc_sh.at[pl.ds(off, SL)],
                  p1_h.at[pl.ds(c * PT + off, SL)])
  plsc.subcore_barrier()

  @pl.when(s == 0)
  def _():
    pl.semaphore_signal(sem_x, 1, core_index=1 - c)
    pl.semaphore_wait(sem_x, 1)

  plsc.subcore_barrier()

  pltpu.sync_copy(p1_h.at[pl.ds((1 - c) * PT + off, SL)], m1)
  pltpu.sync_copy(acc_sh.at[pl.ds(off, SL)], m0)

  @plsc.parallel_loop(0, SL, step=L, unroll=4)
  def t2_body(o):
    q = pl.ds(o, L)
    m = m0[q] + m1[q]
    abuf[q] = _tanh16(m)
    msto[q] = m * scale
  _publish_a(off, acc_sh, a_sh, msto, abuf)
  _edge_phase(wid, src_h, dst_h, w_h, acc_sh, a_sh, a_loc,
              srcb, wb, dstb, valb, sem_in, sem_sc, n_edges, n_chunks)
  plsc.subcore_barrier()

  tail_lo = IN_F + ASSOC - (NS - 1) * SL

  @pl.when(s == NS - 1)
  def _():
    pltpu.sync_copy(acc_sh.at[pl.ds((NS - 1) * SL, SL)], m0)
    pltpu.sync_copy(m0.at[pl.ds(tail_lo, OUT_F)],
                    y_h.at[pl.ds(c * OUT_F, OUT_F)])


def _out_body(t_ref, y_ref):
  y_ref[:] = jnp.tanh(t_ref[0, :] + t_ref[1, :])


@functools.lru_cache(maxsize=None)
def _build(e_as, e_in):
  assert e_in % (NS * L) == 0
  assert e_as % 8 == 0
  assert (e_as // NW) - 8 >= C
  mesh = plsc.VectorSubcoreMesh(core_axis_name="c", subcore_axis_name="s")
  partials = jax.ShapeDtypeStruct((NC * PT,), jnp.float32)
  sc_params = pltpu.CompilerParams(needs_layout_passes=False)

  max_count = 8 * ((e_as // 8 + NW - 1) // NW)
  nch = -(-max_count // C)
  nch = 2 + 3 * (-(-(nch - 2) // 3))

  edge_scratch = [
      tuple(pltpu.VMEM((C,), jnp.int32) for _ in range(3)),
      tuple(pltpu.VMEM((C,), jnp.float32) for _ in range(3)),
      tuple(pltpu.VMEM((C,), jnp.int32) for _ in range(3)),
      tuple(pltpu.VMEM((C,), jnp.float32) for _ in range(3)),
      pltpu.SemaphoreType.DMA((3,)),
      pltpu.SemaphoreType.DMA((3,)),
  ]

  kfused = pl.kernel(
      functools.partial(_fused_body, n_in=e_in, n_edges=e_as, n_chunks=nch),
      out_type=(jax.ShapeDtypeStruct((NC * OUT_F,), jnp.float32), partials),
      mesh=mesh,
      compiler_params=sc_params,
      scratch_types=[
          pltpu.VMEM_SHARED((PT,), jnp.float32),
          pltpu.VMEM_SHARED((PT,), jnp.float32),
          pltpu.VMEM((PT,), jnp.float32),
          pltpu.VMEM((SL,), jnp.float32),
          pltpu.VMEM((SL,), jnp.float32),
          pltpu.VMEM((SL,), jnp.float32),
          pltpu.VMEM((SL,), jnp.float32),
          pltpu.VMEM((IN_F,), jnp.float32),
          pltpu.VMEM((e_in // NS,), jnp.int32),
          pltpu.VMEM((e_in // NS,), jnp.int32),
          pltpu.VMEM((e_in // NS,), jnp.float32),
          pltpu.VMEM((e_in // NS,), jnp.float32),
      ] + edge_scratch + [pltpu.SemaphoreType.REGULAR],
  )

  kout = pl.pallas_call(
      _out_body,
      out_shape=jax.ShapeDtypeStruct((OUT_F,), jnp.float32),
  )
  return kfused, kout


def kernel(x, w_in, w_assoc, src_in, dst_in, src_assoc, dst_assoc):
  kfused, kout = _build(src_assoc.shape[0], src_in.shape[0])
  xv = x.reshape(IN_F)
  tails, _ = kfused(xv, src_in, dst_in, w_in, src_assoc, dst_assoc, w_assoc)
  return kout(tails.reshape(NC, OUT_F))


# speedup vs baseline: 1.1099x; 1.1099x over previous
import functools

import jax
import jax.numpy as jnp
from jax import lax
from jax.experimental import pallas as pl
from jax.experimental.pallas import tpu as pltpu
from jax.experimental.pallas import tpu_sc as plsc

IN_F = 512
ASSOC = 50000
OUT_F = 512
TOTAL = IN_F + ASSOC + OUT_F
NC = 2
NS = 16
NW = NC * NS
L = 16
SL = 3200
PT = NS * SL
C = 4096
CI = 2048


def _tanh16(v):
  e = jnp.exp(v + v)
  return 1.0 - 2.0 / (e + 1.0)


def _edge_phase(wid, src_h, dst_h, w_h, acc_sh, a_sh, a_loc,
                srcb, wb, dstb, valb, sem_in, sem_sc, n_edges, n_chunks):
  e8 = n_edges // 8
  start = 8 * ((e8 * wid) // NW)
  end = 8 * ((e8 * (wid + 1)) // NW)
  iota = lax.iota(jnp.int32, L)

  def chunk_base(k):
    b0 = start + k * C
    b = jnp.minimum(b0, end - C)
    return b, b0 - b

  def in_descs(k, h):
    b, _ = chunk_base(k)
    d = [pltpu.make_async_copy(src_h.at[pl.ds(b, C)], srcb[h],
                               sem_in.at[h]),
         pltpu.make_async_copy(w_h.at[pl.ds(b, C)], wb[h], sem_in.at[h])]
    for j in range(C // CI):
      d.append(pltpu.make_async_copy(dst_h.at[pl.ds(b + j * CI, CI)],
                                     dstb[h][j], sem_in.at[h]))
    return d

  def sc_descs(h):
    return [pltpu.make_async_copy(valb[h].at[pl.ds(j * CI, CI)],
                                  acc_sh.at[dstb[h][j]], sem_sc.at[h])
            for j in range(C // CI)]

  def compute(k, h):
    _, pinv = chunk_base(k)

    @plsc.parallel_loop(0, C, step=L, unroll=4)
    def g_body(o):
      q = pl.ds(o, L)
      sv = srcb[h][q]
      av = plsc.load_gather(a_loc, [sv])
      wv = jnp.where((o + iota) >= pinv, wb[h][q], 0.0)
      valb[h][q] = av * wv

  def run_chunk(k, h, drain_h):
    if drain_h is not None:
      for d in sc_descs(drain_h):
        d.wait()
    for d in in_descs(k + 1, (h + 1) % 3):
      d.start()
    for d in in_descs(k, h):
      d.wait()
    compute(k, h)
    for d in sc_descs(h):
      d.start(add=True)

  a_copy = pltpu.make_async_copy(a_sh, a_loc, sem_in.at[2])
  a_copy.start()
  for d in in_descs(0, 0):
    d.start()
  a_copy.wait()
  run_chunk(0, 0, None)
  run_chunk(1, 1, None)

  def triple_body(p, carry):
    k = 2 + 3 * p
    for h_off in range(3):
      kk = k + h_off
      h = (2 + h_off) % 3
      run_chunk(kk, h, (h + 1) % 3)
    return carry

  lax.fori_loop(0, (n_chunks - 2) // 3, triple_body, 0)

  for d in sc_descs((n_chunks - 2) % 3):
    d.wait()
  for d in sc_descs((n_chunks - 1) % 3):
    d.wait()
  for d in in_descs(n_chunks, n_chunks % 3):
    d.wait()


def _publish_a(off, acc_sh, a_sh, msto, abuf):
  pltpu.sync_copy(msto, acc_sh.at[pl.ds(off, SL)])
  pltpu.sync_copy(abuf, a_sh.at[pl.ds(off, SL)])
  plsc.subcore_barrier()


def _fused_body(x_h, si_h, di_h, wi_h, src_h, dst_h, w_h, y_h, p1_h,
                acc_sh, a_sh, a_loc, m0, m1, msto, abuf, x_loc,
                sib, dib, wib, vib, srcb, wb, dstb, valb,
                sem_in, sem_sc, sem_x,
                n_in, n_edges, n_chunks):
  c = lax.axis_index("c")
  s = lax.axis_index("s")
  wid = s * NC + c
  off = s * SL

  zeros = jnp.zeros((L,), jnp.float32)

  @plsc.parallel_loop(0, SL, step=L, unroll=4)
  def z_body(o):
    msto[pl.ds(o, L)] = zeros
  pltpu.sync_copy(msto, acc_sh.at[pl.ds(off, SL)])
  pltpu.sync_copy(x_h, x_loc)
  plsc.subcore_barrier()

  per = n_in // NS
  base = s * per
  pltpu.sync_copy(si_h.at[pl.ds(base, per)], sib)
  pltpu.sync_copy(wi_h.at[pl.ds(base, per)], wib)
  pltpu.sync_copy(di_h.at[pl.ds(base, per)], dib)

  @plsc.parallel_loop(0, per, step=L, unroll=4)
  def g_body(o):
    q = pl.ds(o, L)
    sv = sib[q]
    av = plsc.load_gather(x_loc, [sv])
    vib[q] = av * wib[q]
  pltpu.sync_copy(vib, acc_sh.at[dib], add=True)
  plsc.subcore_barrier()

  pltpu.sync_copy(acc_sh.at[pl.ds(off, SL)], m0)
  scale = jnp.where(c == 0, 1.0, 0.0).astype(jnp.float32)

  @plsc.parallel_loop(0, SL, step=L, unroll=4)
  def t_body(o):
    q = pl.ds(o, L)
    m = m0[q]
    abuf[q] = _tanh16(m)
    msto[q] = m * scale
  _publish_a(off, acc_sh, a_sh, msto, abuf)
  _edge_phase(wid, src_h, dst_h, w_h, acc_sh, a_sh, a_loc,
              srcb, wb, dstb, valb, sem_in, sem_sc, n_edges, n_chunks)
  plsc.subcore_barrier()

  pltpu.sync_copy(acc_sh.at[pl.ds(off, SL)],
                  p1_h.at[pl.ds(c * PT + off, SL)])
  plsc.subcore_barrier()

  @pl.when(s == 0)
  def _():
    pl.semaphore_signal(sem_x, 1, core_index=1 - c)
    pl.semaphore_wait(sem_x, 1)

  plsc.subcore_barrier()

  pltpu.sync_copy(p1_h.at[pl.ds((1 - c) * PT + off, SL)], m1)
  pltpu.sync_copy(acc_sh.at[pl.ds(off, SL)], m0)

  @plsc.parallel_loop(0, SL, step=L, unroll=4)
  def t2_body(o):
    q = pl.ds(o, L)
    m = m0[q] + m1[q]
    abuf[q] = _tanh16(m)
    msto[q] = m * scale
  _publish_a(off, acc_sh, a_sh, msto, abuf)
  _edge_phase(wid, src_h, dst_h, w_h, acc_sh, a_sh, a_loc,
              srcb, wb, dstb, valb, sem_in, sem_sc, n_edges, n_chunks)
  plsc.subcore_barrier()

  tail_lo = IN_F + ASSOC - (NS - 1) * SL

  @pl.when(s == NS - 1)
  def _():
    pltpu.sync_copy(acc_sh.at[pl.ds((NS - 1) * SL, SL)], m0)
    pltpu.sync_copy(m0.at[pl.ds(tail_lo, OUT_F)],
                    y_h.at[pl.ds(c * OUT_F, OUT_F)])


def _out_body(t_ref, y_ref):
  y_ref[:] = jnp.tanh(t_ref[0, :] + t_ref[1, :])


@functools.lru_cache(maxsize=None)
def _build(e_as, e_in):
  assert e_in % (NS * L) == 0
  assert e_as % 8 == 0
  assert (e_as // NW) - 8 >= C
  mesh = plsc.VectorSubcoreMesh(core_axis_name="c", subcore_axis_name="s")
  partials = jax.ShapeDtypeStruct((NC * PT,), jnp.float32)
  sc_params = pltpu.CompilerParams(needs_layout_passes=False)

  max_count = 8 * ((e_as // 8 + NW - 1) // NW)
  nch = -(-max_count // C)
  nch = 2 + 3 * (-(-(nch - 2) // 3))

  edge_scratch = [
      tuple(pltpu.VMEM((C,), jnp.int32) for _ in range(3)),
      tuple(pltpu.VMEM((C,), jnp.float32) for _ in range(3)),
      tuple(tuple(pltpu.VMEM((CI,), jnp.int32) for _ in range(C // CI))
            for _ in range(3)),
      tuple(pltpu.VMEM((C,), jnp.float32) for _ in range(3)),
      pltpu.SemaphoreType.DMA((3,)),
      pltpu.SemaphoreType.DMA((3,)),
  ]

  kfused = pl.kernel(
      functools.partial(_fused_body, n_in=e_in, n_edges=e_as, n_chunks=nch),
      out_type=(jax.ShapeDtypeStruct((NC * OUT_F,), jnp.float32), partials),
      mesh=mesh,
      compiler_params=sc_params,
      scratch_types=[
          pltpu.VMEM_SHARED((PT,), jnp.float32),
          pltpu.VMEM_SHARED((PT,), jnp.float32),
          pltpu.VMEM((PT,), jnp.float32),
          pltpu.VMEM((SL,), jnp.float32),
          pltpu.VMEM((SL,), jnp.float32),
          pltpu.VMEM((SL,), jnp.float32),
          pltpu.VMEM((SL,), jnp.float32),
          pltpu.VMEM((IN_F,), jnp.float32),
          pltpu.VMEM((e_in // NS,), jnp.int32),
          pltpu.VMEM((e_in // NS,), jnp.int32),
          pltpu.VMEM((e_in // NS,), jnp.float32),
          pltpu.VMEM((e_in // NS,), jnp.float32),
      ] + edge_scratch + [pltpu.SemaphoreType.REGULAR],
  )

  kout = pl.pallas_call(
      _out_body,
      out_shape=jax.ShapeDtypeStruct((OUT_F,), jnp.float32),
  )
  return kfused, kout


def kernel(x, w_in, w_assoc, src_in, dst_in, src_assoc, dst_assoc):
  kfused, kout = _build(src_assoc.shape[0], src_in.shape[0])
  xv = x.reshape(IN_F)
  tails, _ = kfused(xv, src_in, dst_in, w_in, src_assoc, dst_assoc, w_assoc)
  return kout(tails.reshape(NC, OUT_F))
